# Initial kernel scaffold; baseline (speedup 1.0000x reference)
import jax
import jax.numpy as jnp
from jax.experimental import pallas as pl


def kernel(vocab_logits, expected_output_tokens, batch_idx):
    raise NotImplementedError("write your pallas kernel here")


# speedup vs baseline: 3.3581x; 3.3581x over previous
import functools
import math

import jax
import jax.numpy as jnp
from jax import lax
from jax.experimental import pallas as pl
from jax.experimental.pallas import tpu as pltpu

_PAD = 0
_CONF = 0.9


def _body(nblk, bw, V, tok_ref, x_ref, out_ref, acc_ref, acc_le_ref, l0_ref):
    i = pl.program_id(0)
    x = x_ref[...]
    col = i * bw + lax.broadcasted_iota(jnp.int32, x.shape, 1)
    xm = jnp.where(col < V, x, 0.0)
    part = jnp.sum(xm, axis=1, keepdims=True)
    tok = tok_ref[...]
    le_part = jnp.sum(jnp.where(col == tok, x, 0.0), axis=1, keepdims=True)

    @pl.when(i == 0)
    def _():
        acc_ref[...] = jnp.zeros_like(acc_ref)
        acc_le_ref[...] = jnp.zeros_like(acc_le_ref)
        l0_ref[...] = x[:, 0:1]

    acc_ref[:, 0:1] += part
    acc_le_ref[:, 0:1] += le_part

    @pl.when(i == nblk - 1)
    def _():
        d = (1.0 - _CONF) / (V - 2)
        a_const = (V - 2) * d * math.log(d) + _CONF * math.log(_CONF)
        rowsum = acc_ref[:, 0:1]
        le = acc_le_ref[:, 0:1]
        l0 = l0_ref[...]
        nonpad = (tok != _PAD).astype(jnp.float32)
        contrib = d * (rowsum - l0 - le) + _CONF * le
        n = jnp.sum(nonpad)
        tot = jnp.sum(contrib * nonpad)
        loss = (n * a_const - tot) / jnp.maximum(n, 1.0)
        out_ref[...] = jnp.full(out_ref.shape, loss)


def kernel(vocab_logits, expected_output_tokens, batch_idx):
    B, S, V = vocab_logits.shape
    R = B * S
    x2 = vocab_logits.reshape(R, V)
    tok2 = expected_output_tokens.reshape(R, 1)
    bw = 6272
    nblk = pl.cdiv(V, bw)
    out = pl.pallas_call(
        functools.partial(_body, nblk, bw, V),
        grid=(nblk,),
        in_specs=[
            pl.BlockSpec((R, 1), lambda i: (0, 0)),
            pl.BlockSpec((R, bw), lambda i: (0, i)),
        ],
        out_specs=pl.BlockSpec((8, 128), lambda i: (0, 0)),
        out_shape=jax.ShapeDtypeStruct((8, 128), jnp.float32),
        scratch_shapes=[
            pltpu.VMEM((R, 128), jnp.float32),
            pltpu.VMEM((R, 128), jnp.float32),
            pltpu.VMEM((R, 1), jnp.float32),
        ],
    )(tok2, x2)
    return out[0, 0]
